# baseline (device time: 28357 ns/iter reference)
import jax
import jax.numpy as jnp
from jax import lax
from jax.experimental import pallas as pl
from jax.experimental.pallas import tpu as pltpu

N_DEV = 4
N_Q = 4

S_DIRECT, S_FAR, S_COMB, S_AGOWN = 0, 1, 2, 3
R_DIRECT, R_FAR, R_COMB, R_AG_L, R_AG_R, R_AGFAR = 4, 5, 6, 7, 8, 9
N_SLOT = 10

T_DIRECT, T_FAR, T_COMB, T_AG_R, T_AG_L, T_AGFAR = 0, 1, 2, 3, 4, 5
N_TYPE = 6


def kernel(A, B):
    m, k = A.shape
    _, n = B.shape
    ch = m // N_DEV
    qw = n // N_Q

    f32 = jnp.float32
    bf16 = jnp.bfloat16

    QDIR = (0, 0, 1, 1)

    def body(a_ref, b_ref, out_ref, a_vmem, b_vmem, out_stage, p_ref, comm,
             far_stage, direct_stage, send_sems, recv_sems, in_sems,
             out_dma_sems):
        my = lax.axis_index("i")
        left = (my - 1) % N_DEV
        right = (my + 1) % N_DEV

        def rows_of(c):
            return pl.ds((c % N_DEV) * ch, ch)

        copy_b = pltpu.make_async_copy(b_ref, b_vmem, in_sems.at[0])
        copy_b.start()
        a_order = (my + 2, my + 1, my - 1, my)
        a_copies = []
        for i, c in enumerate(a_order):
            cp = pltpu.make_async_copy(
                a_ref.at[rows_of(c), :], a_vmem.at[rows_of(c), :],
                in_sems.at[1 + i])
            cp.start()
            a_copies.append(cp)

        barrier_sem = pltpu.get_barrier_semaphore()
        for nbr in [left, right]:
            pl.semaphore_signal(
                barrier_sem, inc=1,
                device_id=(nbr,), device_id_type=pl.DeviceIdType.MESH,
            )
        pl.semaphore_wait(barrier_sem, 2)
        copy_b.wait()

        def rows(c):
            return pl.ds((c % N_DEV) * ch, ch)

        def cols(q):
            return pl.ds(q * qw, qw)

        def rdma(q, t, src_slot, dst_slot, to_right):
            return pltpu.make_async_remote_copy(
                src_ref=comm.at[q, src_slot],
                dst_ref=comm.at[q, dst_slot],
                send_sem=send_sems.at[q, t],
                recv_sem=recv_sems.at[q, t],
                device_id=(right if to_right else left,),
                device_id_type=pl.DeviceIdType.MESH,
            )

        dirs = {}
        for q in range(N_Q):
            r = QDIR[q] == 0
            dirs[(q, T_DIRECT)] = pltpu.make_async_remote_copy(
                src_ref=direct_stage.at[QDIR[q], :, pl.ds((q % 2) * qw, qw)],
                dst_ref=comm.at[q, R_DIRECT],
                send_sem=send_sems.at[q, T_DIRECT],
                recv_sem=recv_sems.at[q, T_DIRECT],
                device_id=(right if r else left,),
                device_id_type=pl.DeviceIdType.MESH,
            )
            dirs[(q, T_FAR)] = pltpu.make_async_remote_copy(
                src_ref=far_stage.at[:, cols(q)],
                dst_ref=comm.at[q, R_FAR],
                send_sem=send_sems.at[q, T_FAR],
                recv_sem=recv_sems.at[q, T_FAR],
                device_id=(left if r else right,),
                device_id_type=pl.DeviceIdType.MESH,
            )
            dirs[(q, T_COMB)] = rdma(q, T_COMB, S_COMB, R_COMB, not r)
            dirs[(q, T_AG_R)] = rdma(q, T_AG_R, S_AGOWN, R_AG_L, True)
            dirs[(q, T_AG_L)] = rdma(q, T_AG_L, S_AGOWN, R_AG_R, False)
            dirs[(q, T_AGFAR)] = rdma(
                q, T_AGFAR, R_AG_L if r else R_AG_R, R_AGFAR, r)

        def dot_block(c):
            p_ref[rows(c), :] = jnp.dot(
                a_vmem[rows(c), :], b_vmem[...], preferred_element_type=f32)

        out_copies = []

        def put_out(c, q, kind, values):
            out_stage[rows(c), cols(q)] = values
            cp = pltpu.make_async_copy(
                out_stage.at[rows(c), cols(q)],
                out_ref.at[rows(c), cols(q)],
                out_dma_sems.at[q, kind],
            )
            cp.start()
            out_copies.append(cp)

        def direct_chunk(q):
            return my + 1 if QDIR[q] == 0 else my - 1

        def relay_chunk(q):
            return my - 1 if QDIR[q] == 0 else my + 1

        a_copies[0].wait()
        dot_block(my + 2)
        far_stage[...] = p_ref[rows(my + 2), :].astype(bf16)
        for q in (0, 2, 1, 3):
            dirs[(q, T_FAR)].start()

        a_copies[1].wait()
        dot_block(my + 1)
        direct_stage[0] = p_ref[rows(my + 1), pl.ds(0, 2 * qw)].astype(bf16)
        dirs[(0, T_DIRECT)].start()
        a_copies[2].wait()
        dot_block(my - 1)
        direct_stage[1] = p_ref[rows(my - 1), pl.ds(2 * qw, 2 * qw)].astype(bf16)
        dirs[(2, T_DIRECT)].start()
        a_copies[3].wait()
        dot_block(my)

        for qs in ((0, 2), (1, 3)):
            for q in qs:
                dirs[(q, T_FAR)].wait_recv()
                comm[q, S_COMB] = (
                    comm[q, R_FAR].astype(f32)
                    + p_ref[rows(relay_chunk(q)), cols(q)]
                ).astype(bf16)
                dirs[(q, T_COMB)].start()
            if qs == (0, 2):
                dirs[(1, T_DIRECT)].start()
                dirs[(3, T_DIRECT)].start()

        ag_full = {}
        for qs, feeds_only in (((0, 2), False), ((1, 3), True)):
            for q in qs:
                dirs[(q, T_DIRECT)].wait_recv()
                dirs[(q, T_COMB)].wait_recv()
                full = jnp.maximum(
                    p_ref[rows(my), cols(q)]
                    + comm[q, R_DIRECT].astype(f32)
                    + comm[q, R_COMB].astype(f32), 0.0)
                comm[q, S_AGOWN] = full.astype(bf16)
                ag_full[q] = full
            for q in qs:
                dirs[(q, T_AG_R if QDIR[q] == 0 else T_AG_L)].start()
            if not feeds_only:
                for q in qs:
                    dirs[(q, T_AG_L if QDIR[q] == 0 else T_AG_R)].start()
            for q in qs:
                put_out(my, q, 0, ag_full[q])

        for q in (0, 2):
            src = R_AG_L if QDIR[q] == 0 else R_AG_R
            dirs[(q, T_AG_R if QDIR[q] == 0 else T_AG_L)].wait_recv()
            dirs[(q, T_AGFAR)].start()
            put_out(relay_chunk(q), q, 1, comm[q, src].astype(f32))

        for q in (1, 3):
            dirs[(q, T_AG_L if QDIR[q] == 0 else T_AG_R)].start()

        for q in (1, 3):
            src = R_AG_L if QDIR[q] == 0 else R_AG_R
            dirs[(q, T_AG_R if QDIR[q] == 0 else T_AG_L)].wait_recv()
            dirs[(q, T_AGFAR)].start()
            put_out(relay_chunk(q), q, 1, comm[q, src].astype(f32))

        for q in (0, 2, 1, 3):
            src = R_AG_R if QDIR[q] == 0 else R_AG_L
            dirs[(q, T_AG_L if QDIR[q] == 0 else T_AG_R)].wait_recv()
            put_out(direct_chunk(q), q, 2, comm[q, src].astype(f32))
        for q in (0, 2, 1, 3):
            dirs[(q, T_AGFAR)].wait_recv()
            put_out(my + 2, q, 3, comm[q, R_AGFAR].astype(f32))

        for r in dirs.values():
            r.wait_send()
        for cp in out_copies:
            cp.wait()

    return pl.pallas_call(
        body,
        out_shape=jax.ShapeDtypeStruct((m, n), f32),
        in_specs=[
            pl.BlockSpec(memory_space=pl.ANY),
            pl.BlockSpec(memory_space=pl.ANY),
        ],
        out_specs=pl.BlockSpec(memory_space=pl.ANY),
        scratch_shapes=[
            pltpu.VMEM((m, k), f32),
            pltpu.VMEM((k, n), f32),
            pltpu.VMEM((m, n), f32),
            pltpu.VMEM((m, n), f32),
            pltpu.VMEM((N_Q, N_SLOT, ch, qw), bf16),
            pltpu.VMEM((ch, n), bf16),
            pltpu.VMEM((2, ch, n // 2), bf16),
            pltpu.SemaphoreType.DMA((N_Q, N_TYPE)),
            pltpu.SemaphoreType.DMA((N_Q, N_TYPE)),
            pltpu.SemaphoreType.DMA((5,)),
            pltpu.SemaphoreType.DMA((N_Q, 4)),
        ],
        compiler_params=pltpu.CompilerParams(collective_id=0),
    )(A, B)
